# Initial kernel scaffold; baseline (speedup 1.0000x reference)
import functools

import jax
import jax.numpy as jnp
from jax import lax
from jax.experimental import pallas as pl
from jax.experimental.pallas import tpu as pltpu
from jax.experimental.pallas import tpu_sc as plsc

_N = 10000
_E = 320000
_H = 128
_G = 256
_L = 3

_NC = 2
_NS = 16
_NW = _NC * _NS

_CH = 128
_CPW = -(-_E // (_NW * _CH))
_EPW = _CPW * _CH
_EP = _EPW * _NW

_NACC = 10016
_ZSTR = _NACC // _NS
_OSTR = _N // _NS
_DW = 16

_NZP = 10240
_ZB = 64
_ZPW = _NZP // _NW
_ZC = _ZPW // _ZB

_mesh = plsc.VectorSubcoreMesh(core_axis_name="c", subcore_axis_name="s")


@functools.partial(
    pl.kernel,
    out_type=jax.ShapeDtypeStruct((_NZP, _H), jnp.float32),
    mesh=_mesh,
    scratch_types=[
        pltpu.VMEM((_ZB,), jnp.int32),
        pltpu.VMEM((_ZB, _H), jnp.float32),
        pltpu.SemaphoreType.DMA,
    ],
)
def _zemb_gather(table_hbm, z_hbm, out_hbm, idx_v, rows_v, sem):
    wid = lax.axis_index("c") * _NS + lax.axis_index("s")
    base = wid * _ZPW

    def chunk(c, carry):
        off = base + c * _ZB
        pltpu.sync_copy(z_hbm.at[pl.ds(off, _ZB)], idx_v)
        pltpu.async_copy(table_hbm.at[idx_v], rows_v, sem).wait()
        pltpu.sync_copy(rows_v, out_hbm.at[pl.ds(off, _ZB)])
        return carry

    lax.fori_loop(0, _ZC, chunk, 0)


def _make_segsum(with_deg):
    out_type = [jax.ShapeDtypeStruct((_NC, _N, _H), jnp.float32)]
    scratch = [
        pltpu.VMEM((_CH,), jnp.int32),
        pltpu.VMEM((_CH,), jnp.int32),
        pltpu.VMEM((_CH, _H), jnp.float32),
        pltpu.VMEM_SHARED((_NACC, _H), jnp.float32),
        pltpu.SemaphoreType.DMA,
    ]
    if with_deg:
        out_type.append(jax.ShapeDtypeStruct((_NC, _N, _DW), jnp.float32))
        scratch += [
            pltpu.VMEM((_CH, _DW), jnp.float32),
            pltpu.VMEM_SHARED((_NACC, _DW), jnp.float32),
        ]

    def body(refs):
        if with_deg:
            (y_hbm, src_hbm, dst_hbm, zeros_hbm, zeros16_hbm, ones_hbm,
             out_hbm, deg_hbm, src_v, dst_v, rows_v, acc, sem, ones_v, dacc) = refs
        else:
            (y_hbm, src_hbm, dst_hbm, zeros_hbm,
             out_hbm, src_v, dst_v, rows_v, acc, sem) = refs
        cid = lax.axis_index("c")
        sid = lax.axis_index("s")
        wid = cid * _NS + sid

        pltpu.sync_copy(zeros_hbm.at[pl.ds(sid * _ZSTR, _ZSTR)],
                        acc.at[pl.ds(sid * _ZSTR, _ZSTR)])
        if with_deg:
            pltpu.sync_copy(zeros16_hbm.at[pl.ds(sid * _ZSTR, _ZSTR)],
                            dacc.at[pl.ds(sid * _ZSTR, _ZSTR)])
            pltpu.sync_copy(ones_hbm, ones_v)
        plsc.subcore_barrier()

        def chunk(c, carry):
            pltpu.sync_copy(src_hbm.at[wid, c], src_v)
            pltpu.sync_copy(dst_hbm.at[wid, c], dst_v)
            pltpu.async_copy(y_hbm.at[src_v], rows_v, sem).wait()
            pltpu.sync_copy(rows_v, acc.at[dst_v], add=True)
            if with_deg:
                pltpu.sync_copy(ones_v, dacc.at[dst_v], add=True)
            return carry

        lax.fori_loop(0, _CPW, chunk, 0)
        plsc.subcore_barrier()

        pltpu.sync_copy(acc.at[pl.ds(sid * _OSTR, _OSTR)],
                        out_hbm.at[cid, pl.ds(sid * _OSTR, _OSTR)])
        if with_deg:
            pltpu.sync_copy(dacc.at[pl.ds(sid * _OSTR, _OSTR)],
                            deg_hbm.at[cid, pl.ds(sid * _OSTR, _OSTR)])

    def fn(*refs):
        return body(refs)

    return pl.kernel(fn, out_type=out_type, mesh=_mesh, scratch_types=scratch)


_segsum_deg = _make_segsum(True)
_segsum = _make_segsum(False)


_NBP = 10016


@functools.partial(
    pl.kernel,
    out_type=(jax.ShapeDtypeStruct((_G, _H), jnp.float32),
              jax.ShapeDtypeStruct((_G, _H), jnp.float32)),
    mesh=_mesh,
    scratch_types=[
        pltpu.VMEM((_NBP,), jnp.int32),
        pltpu.VMEM((16,), jnp.int32),
        pltpu.VMEM((16, _H), jnp.float32),
        pltpu.SemaphoreType.DMA,
    ],
)
def _head_gather(x3_hbm, batch_hbm, xs_hbm, xd_hbm, batch_v, idx_v, rows_v, sem):
    wid = lax.axis_index("c") * _NS + lax.axis_index("s")
    side = wid // 16
    gchunk = wid % 16

    pltpu.sync_copy(batch_hbm, batch_v)

    g16 = gchunk * 16 + lax.iota(jnp.int32, (16,), 0)
    lo = jnp.zeros((16,), jnp.int32)
    hi = jnp.full((16,), _N, jnp.int32)

    def step(_, carry):
        lo, hi = carry
        mid = (lo + hi) // 2
        vals = plsc.load_gather(batch_v, [mid])
        go = lo < hi
        pred = vals < g16
        lo = jnp.where(go & pred, mid + 1, lo)
        hi = jnp.where(go & (~pred), mid, hi)
        return lo, hi

    lo, _ = lax.fori_loop(0, 14, step, (lo, hi))
    idx = jnp.minimum(lo + side, _N - 1)
    idx_v[...] = idx
    pltpu.async_copy(x3_hbm.at[idx_v], rows_v, sem).wait()

    @pl.when(side == 0)
    def _():
        pltpu.sync_copy(rows_v, xs_hbm.at[pl.ds(gchunk * 16, 16)])

    @pl.when(side == 1)
    def _():
        pltpu.sync_copy(rows_v, xd_hbm.at[pl.ds(gchunk * 16, 16)])


_RB = 1000
_NG = _N // _RB


def _mm2_body(x_ref, wl_ref, wr_ref, bl_ref, y_ref, r_ref):
    x = x_ref[...]
    y_ref[...] = jnp.dot(x, wl_ref[...], preferred_element_type=jnp.float32)
    r_ref[...] = (jnp.dot(x, wr_ref[...], preferred_element_type=jnp.float32)
                  + bl_ref[...])


def _mm2(x, wl, wr, bl):
    return pl.pallas_call(
        _mm2_body,
        grid=(_NG,),
        in_specs=[
            pl.BlockSpec((_RB, _H), lambda i: (i, 0)),
            pl.BlockSpec((_H, _H), lambda i: (0, 0)),
            pl.BlockSpec((_H, _H), lambda i: (0, 0)),
            pl.BlockSpec((1, _H), lambda i: (0, 0)),
        ],
        out_specs=[
            pl.BlockSpec((_RB, _H), lambda i: (i, 0)),
            pl.BlockSpec((_RB, _H), lambda i: (i, 0)),
        ],
        out_shape=[
            jax.ShapeDtypeStruct((_N, _H), jnp.float32),
            jax.ShapeDtypeStruct((_N, _H), jnp.float32),
        ],
    )(x, wl, wr, bl)


def _combine(p0, p1, d0, d1, r):
    deg = jnp.maximum(d0 + d1, 1.0)
    inv = 1.0 / deg
    invb = jnp.concatenate([inv] * (_H // _DW), axis=1)
    return (p0 + p1) * invb + r


def _combine_mm_body(p0_ref, p1_ref, d0_ref, d1_ref, r_ref,
                     wl_ref, wr_ref, bl_ref, y_ref, rn_ref):
    x = _combine(p0_ref[...], p1_ref[...], d0_ref[...], d1_ref[...], r_ref[...])
    x = jnp.maximum(x, 0.0)
    y_ref[...] = jnp.dot(x, wl_ref[...], preferred_element_type=jnp.float32)
    rn_ref[...] = (jnp.dot(x, wr_ref[...], preferred_element_type=jnp.float32)
                   + bl_ref[...])


def _combine_mm(p0, p1, d0, d1, r, wl, wr, bl):
    return pl.pallas_call(
        _combine_mm_body,
        grid=(_NG,),
        in_specs=[
            pl.BlockSpec((_RB, _H), lambda i: (i, 0)),
            pl.BlockSpec((_RB, _H), lambda i: (i, 0)),
            pl.BlockSpec((_RB, _DW), lambda i: (i, 0)),
            pl.BlockSpec((_RB, _DW), lambda i: (i, 0)),
            pl.BlockSpec((_RB, _H), lambda i: (i, 0)),
            pl.BlockSpec((_H, _H), lambda i: (0, 0)),
            pl.BlockSpec((_H, _H), lambda i: (0, 0)),
            pl.BlockSpec((1, _H), lambda i: (0, 0)),
        ],
        out_specs=[
            pl.BlockSpec((_RB, _H), lambda i: (i, 0)),
            pl.BlockSpec((_RB, _H), lambda i: (i, 0)),
        ],
        out_shape=[
            jax.ShapeDtypeStruct((_N, _H), jnp.float32),
            jax.ShapeDtypeStruct((_N, _H), jnp.float32),
        ],
    )(p0, p1, d0, d1, r, wl, wr, bl)


def _final_combine_body(p0_ref, p1_ref, d0_ref, d1_ref, r_ref, x_ref):
    x_ref[...] = _combine(p0_ref[...], p1_ref[...], d0_ref[...], d1_ref[...],
                          r_ref[...])


def _final_combine(p0, p1, d0, d1, r):
    return pl.pallas_call(
        _final_combine_body,
        grid=(_NG,),
        in_specs=[
            pl.BlockSpec((_RB, _H), lambda i: (i, 0)),
            pl.BlockSpec((_RB, _H), lambda i: (i, 0)),
            pl.BlockSpec((_RB, _DW), lambda i: (i, 0)),
            pl.BlockSpec((_RB, _DW), lambda i: (i, 0)),
            pl.BlockSpec((_RB, _H), lambda i: (i, 0)),
        ],
        out_specs=pl.BlockSpec((_RB, _H), lambda i: (i, 0)),
        out_shape=jax.ShapeDtypeStruct((_N, _H), jnp.float32),
    )(p0, p1, d0, d1, r)


def _head_mlp_body(xs_ref, xd_ref, w1_ref, b1_ref, w2_ref, b2_ref, o_ref):
    h = jnp.dot(xs_ref[...] * xd_ref[...], w1_ref[...],
                preferred_element_type=jnp.float32) + b1_ref[...]
    h = jnp.maximum(h, 0.0)
    o_ref[...] = jnp.dot(h, w2_ref[...],
                         preferred_element_type=jnp.float32) + b2_ref[...]


def _head_mlp(xs, xd, w1, b1, w2p, b2p):
    return pl.pallas_call(
        _head_mlp_body,
        out_shape=jax.ShapeDtypeStruct((_G, _H), jnp.float32),
    )(xs, xd, w1, b1, w2p, b2p)


def kernel(z, edge_index, batch, z_emb_table, Wl, bl, Wr,
           lin1_W, lin1_b, lin2_W, lin2_b):
    z = z.astype(jnp.int32)
    src = edge_index[0].astype(jnp.int32)
    dst = edge_index[1].astype(jnp.int32)
    batch = batch.astype(jnp.int32)

    npad = _EP - _E
    src_p = jnp.concatenate([src, jnp.zeros((npad,), jnp.int32)])
    src_p = src_p.reshape(_NW, _CPW, _CH)
    dst_p = jnp.concatenate([dst, jnp.full((npad,), _N, jnp.int32)])
    dst_p = dst_p.reshape(_NW, _CPW, _CH)
    z_p = jnp.concatenate([z, jnp.zeros((_NZP - _N,), jnp.int32)])
    batch_p = jnp.concatenate([batch, jnp.zeros((_NBP - _N,), jnp.int32)])

    zeros_h = jnp.zeros((_NACC, _H), jnp.float32)
    zeros16 = jnp.zeros((_NACC, _DW), jnp.float32)
    ones16 = jnp.ones((_CH, _DW), jnp.float32)

    x0 = _zemb_gather(z_emb_table, z_p)
    y, r = _mm2(x0[:_N], Wl[0], Wr[0], bl[0].reshape(1, _H))

    p, dg = _segsum_deg(y, src_p, dst_p, zeros_h, zeros16, ones16)
    y, r = _combine_mm(p[0], p[1], dg[0], dg[1], r,
                       Wl[1], Wr[1], bl[1].reshape(1, _H))

    p = _segsum(y, src_p, dst_p, zeros_h)
    y, r = _combine_mm(p[0], p[1], dg[0], dg[1], r,
                       Wl[2], Wr[2], bl[2].reshape(1, _H))

    p = _segsum(y, src_p, dst_p, zeros_h)
    x3 = _final_combine(p[0], p[1], dg[0], dg[1], r)

    xs, xd = _head_gather(x3, batch_p)
    w2p = jnp.pad(lin2_W, ((0, 0), (0, _H - 1)))
    b2p = jnp.pad(lin2_b, (0, _H - 1)).reshape(1, _H)
    outp = _head_mlp(xs, xd, lin1_W, lin1_b.reshape(1, _H), w2p, b2p)
    return outp[:, :1]


# speedup vs baseline: 3.8715x; 3.8715x over previous
import functools

import jax
import jax.numpy as jnp
from jax import lax
from jax.experimental import pallas as pl
from jax.experimental.pallas import tpu as pltpu
from jax.experimental.pallas import tpu_sc as plsc

_N = 10000
_E = 320000
_H = 128
_G = 256

_NC = 2
_NS = 16
_NW = _NC * _NS

_CH = 128
_CPW = -(-_E // (_NW * _CH))
_EP = _CPW * _CH * _NW
_CHALF = 40

_NACC = 10112
_STR = _NACC // _NS
_DW = 16

_NZP = 10240
_ZB = 64
_ZPW = _NZP // _NW
_ZC = _ZPW // _ZB

_HACC = 384
_HSTR = _HACC // _NS
_NBP = 10240
_BCW = 5

_mesh = plsc.VectorSubcoreMesh(core_axis_name="c", subcore_axis_name="s",
                               num_cores=_NC, num_subcores=_NS)


@functools.partial(
    pl.kernel,
    out_type=jax.ShapeDtypeStruct((_NZP, _H), jnp.float32),
    mesh=_mesh,
    scratch_types=[
        pltpu.VMEM((_ZB,), jnp.int32),
        pltpu.VMEM((_ZB, _H), jnp.float32),
        pltpu.SemaphoreType.DMA,
    ],
)
def _zemb_gather(table_hbm, z_hbm, out_hbm, idx_v, rows_v, sem):
    wid = lax.axis_index("c") * _NS + lax.axis_index("s")
    base = wid * _ZPW

    def chunk(c, carry):
        off = base + c * _ZB
        pltpu.sync_copy(z_hbm.at[pl.ds(off, _ZB)], idx_v)
        pltpu.async_copy(table_hbm.at[idx_v], rows_v, sem).wait()
        pltpu.sync_copy(rows_v, out_hbm.at[pl.ds(off, _ZB)])
        return carry

    lax.fori_loop(0, _ZC, chunk, 0)


@functools.partial(
    pl.kernel,
    out_type=jax.ShapeDtypeStruct((_NC, _NACC, _H), jnp.float32),
    mesh=_mesh,
    scratch_types=[
        pltpu.VMEM((_CHALF, _CH), jnp.int32),
        pltpu.VMEM((_CHALF, _CH), jnp.int32),
        pltpu.VMEM((_CH, _H), jnp.float32),
        pltpu.VMEM_SHARED((_NACC, _H), jnp.float32),
        pltpu.SemaphoreType.DMA,
    ],
)
def _segsum(y_hbm, src_hbm, dst_hbm, zeros_hbm, out_hbm,
            src_v, dst_v, rows_v, acc, sem):
    cid = lax.axis_index("c")
    sid = lax.axis_index("s")
    wid = cid * _NS + sid

    pltpu.sync_copy(zeros_hbm.at[pl.ds(sid * _STR, _STR)],
                    acc.at[pl.ds(sid * _STR, _STR)])
    plsc.subcore_barrier()

    def chunk(c, carry):
        pltpu.async_copy(y_hbm.at[src_v.at[c]], rows_v, sem).wait()
        pltpu.sync_copy(rows_v, acc.at[dst_v.at[c]], add=True)
        return carry

    for half in range(2):
        n_c = _CHALF if half == 0 else _CPW - _CHALF
        pltpu.sync_copy(src_hbm.at[wid, pl.ds(half * _CHALF, n_c)],
                        src_v.at[pl.ds(0, n_c)])
        pltpu.sync_copy(dst_hbm.at[wid, pl.ds(half * _CHALF, n_c)],
                        dst_v.at[pl.ds(0, n_c)])
        lax.fori_loop(0, n_c, chunk, 0)
    plsc.subcore_barrier()

    pltpu.sync_copy(acc.at[pl.ds(sid * _STR, _STR)],
                    out_hbm.at[cid, pl.ds(sid * _STR, _STR)])


@functools.partial(
    pl.kernel,
    out_type=(jax.ShapeDtypeStruct((_NC, _NACC, _H), jnp.float32),
              jax.ShapeDtypeStruct((_G, _H), jnp.float32)),
    mesh=_mesh,
    scratch_types=[
        pltpu.VMEM((_CHALF, _CH), jnp.int32),
        pltpu.VMEM((_BCW, _CH), jnp.int32),
        pltpu.VMEM((_CH, _H), jnp.float32),
        pltpu.VMEM_SHARED((_NACC, _H), jnp.float32),
        pltpu.VMEM_SHARED((_HACC, _H), jnp.float32),
    ],
)
def _deg_hist(dst_hbm, batch_hbm, zeros16_hbm, ones_hbm, deg_hbm, h_hbm,
              dst_v, bidx_v, ones_v, dacc, hacc):
    cid = lax.axis_index("c")
    sid = lax.axis_index("s")
    wid = cid * _NS + sid

    pltpu.sync_copy(zeros16_hbm.at[pl.ds(sid * _STR, _STR)],
                    dacc.at[pl.ds(sid * _STR, _STR)])
    pltpu.sync_copy(ones_hbm, ones_v)

    @pl.when(cid == 0)
    def _():
        pltpu.sync_copy(zeros16_hbm.at[pl.ds(sid * _HSTR, _HSTR)],
                        hacc.at[pl.ds(sid * _HSTR, _HSTR)])

    plsc.subcore_barrier()

    def dchunk(c, carry):
        pltpu.sync_copy(ones_v, dacc.at[dst_v.at[c]], add=True)
        return carry

    for half in range(2):
        n_c = _CHALF if half == 0 else _CPW - _CHALF
        pltpu.sync_copy(dst_hbm.at[wid, pl.ds(half * _CHALF, n_c)],
                        dst_v.at[pl.ds(0, n_c)])
        lax.fori_loop(0, n_c, dchunk, 0)

    @pl.when(cid == 0)
    def _():
        pltpu.sync_copy(batch_hbm.at[sid], bidx_v)

        def hchunk(c, carry):
            pltpu.sync_copy(ones_v, hacc.at[bidx_v.at[c]], add=True)
            return carry

        lax.fori_loop(0, _BCW, hchunk, 0)

    plsc.subcore_barrier()

    pltpu.sync_copy(dacc.at[pl.ds(sid * _STR, _STR)],
                    deg_hbm.at[cid, pl.ds(sid * _STR, _STR)])

    @pl.when(cid == 0)
    def _():
        pltpu.sync_copy(hacc.at[pl.ds(sid * _DW, _DW)],
                        h_hbm.at[pl.ds(sid * _DW, _DW)])


@functools.partial(
    pl.kernel,
    out_type=(jax.ShapeDtypeStruct((_G, _H), jnp.float32),
              jax.ShapeDtypeStruct((_G, _H), jnp.float32)),
    mesh=_mesh,
    scratch_types=[
        pltpu.VMEM((16,), jnp.int32),
        pltpu.VMEM((16, _H), jnp.float32),
        pltpu.SemaphoreType.DMA,
    ],
)
def _center_gather(x3_hbm, cis_hbm, cid_hbm, xs_hbm, xd_hbm, idx_v, rows_v, sem):
    wid = lax.axis_index("c") * _NS + lax.axis_index("s")
    side = wid // 16
    gchunk = wid % 16

    @pl.when(side == 0)
    def _():
        pltpu.sync_copy(cis_hbm.at[pl.ds(gchunk * 16, 16)], idx_v)

    @pl.when(side == 1)
    def _():
        pltpu.sync_copy(cid_hbm.at[pl.ds(gchunk * 16, 16)], idx_v)

    pltpu.async_copy(x3_hbm.at[idx_v], rows_v, sem).wait()

    @pl.when(side == 0)
    def _():
        pltpu.sync_copy(rows_v, xs_hbm.at[pl.ds(gchunk * 16, 16)])

    @pl.when(side == 1)
    def _():
        pltpu.sync_copy(rows_v, xd_hbm.at[pl.ds(gchunk * 16, 16)])


_RB = 1000
_NG = _N // _RB


def _mm2_body(x_ref, wl_ref, wr_ref, bl_ref, y_ref, r_ref):
    x = x_ref[...]
    y_ref[...] = jnp.dot(x, wl_ref[...], preferred_element_type=jnp.float32)
    r_ref[...] = (jnp.dot(x, wr_ref[...], preferred_element_type=jnp.float32)
                  + bl_ref[...])


def _mm2(x, wl, wr, bl):
    return pl.pallas_call(
        _mm2_body,
        grid=(_NG,),
        in_specs=[
            pl.BlockSpec((_RB, _H), lambda i: (i, 0)),
            pl.BlockSpec((_H, _H), lambda i: (0, 0)),
            pl.BlockSpec((_H, _H), lambda i: (0, 0)),
            pl.BlockSpec((1, _H), lambda i: (0, 0)),
        ],
        out_specs=[
            pl.BlockSpec((_RB, _H), lambda i: (i, 0)),
            pl.BlockSpec((_RB, _H), lambda i: (i, 0)),
        ],
        out_shape=[
            jax.ShapeDtypeStruct((_N, _H), jnp.float32),
            jax.ShapeDtypeStruct((_N, _H), jnp.float32),
        ],
    )(x, wl, wr, bl)


def _combine(p0, p1, d0, d1, r):
    inv = 1.0 / jnp.maximum(d0 + d1, 1.0)
    return (p0 + p1) * inv + r


def _combine_mm_body(p0_ref, p1_ref, d0_ref, d1_ref, r_ref,
                     wl_ref, wr_ref, bl_ref, y_ref, rn_ref):
    x = _combine(p0_ref[...], p1_ref[...], d0_ref[...], d1_ref[...], r_ref[...])
    x = jnp.maximum(x, 0.0)
    y_ref[...] = jnp.dot(x, wl_ref[...], preferred_element_type=jnp.float32)
    rn_ref[...] = (jnp.dot(x, wr_ref[...], preferred_element_type=jnp.float32)
                   + bl_ref[...])


def _combine_mm(p0, p1, d0, d1, r, wl, wr, bl):
    return pl.pallas_call(
        _combine_mm_body,
        grid=(_NG,),
        in_specs=[
            pl.BlockSpec((_RB, _H), lambda i: (i, 0)),
            pl.BlockSpec((_RB, _H), lambda i: (i, 0)),
            pl.BlockSpec((_RB, _H), lambda i: (i, 0)),
            pl.BlockSpec((_RB, _H), lambda i: (i, 0)),
            pl.BlockSpec((_RB, _H), lambda i: (i, 0)),
            pl.BlockSpec((_H, _H), lambda i: (0, 0)),
            pl.BlockSpec((_H, _H), lambda i: (0, 0)),
            pl.BlockSpec((1, _H), lambda i: (0, 0)),
        ],
        out_specs=[
            pl.BlockSpec((_RB, _H), lambda i: (i, 0)),
            pl.BlockSpec((_RB, _H), lambda i: (i, 0)),
        ],
        out_shape=[
            jax.ShapeDtypeStruct((_N, _H), jnp.float32),
            jax.ShapeDtypeStruct((_N, _H), jnp.float32),
        ],
    )(p0, p1, d0, d1, r, wl, wr, bl)


def _final_combine_body(p0_ref, p1_ref, d0_ref, d1_ref, r_ref, x_ref):
    x_ref[...] = _combine(p0_ref[...], p1_ref[...], d0_ref[...], d1_ref[...],
                          r_ref[...])


def _final_combine(p0, p1, d0, d1, r):
    return pl.pallas_call(
        _final_combine_body,
        grid=(_NG,),
        in_specs=[
            pl.BlockSpec((_RB, _H), lambda i: (i, 0)),
            pl.BlockSpec((_RB, _H), lambda i: (i, 0)),
            pl.BlockSpec((_RB, _H), lambda i: (i, 0)),
            pl.BlockSpec((_RB, _H), lambda i: (i, 0)),
            pl.BlockSpec((_RB, _H), lambda i: (i, 0)),
        ],
        out_specs=pl.BlockSpec((_RB, _H), lambda i: (i, 0)),
        out_shape=jax.ShapeDtypeStruct((_N, _H), jnp.float32),
    )(p0, p1, d0, d1, r)


def _center_idx_body(h_ref, cis_ref, cid_ref):
    h = h_ref[...]
    row = lax.broadcasted_iota(jnp.int32, (_G, _G), 0)
    col = lax.broadcasted_iota(jnp.int32, (_G, _G), 1)
    m = (col < row).astype(jnp.float32)
    ci = jnp.dot(m, h, preferred_element_type=jnp.float32)
    cii = ci.astype(jnp.int32)
    cis_ref[...] = jnp.minimum(cii, _N - 1)
    cid_ref[...] = jnp.minimum(cii + 1, _N - 1)


def _center_idx(h):
    return pl.pallas_call(
        _center_idx_body,
        out_shape=[
            jax.ShapeDtypeStruct((_G, _H), jnp.int32),
            jax.ShapeDtypeStruct((_G, _H), jnp.int32),
        ],
    )(h)


def _head_mlp_body(xs_ref, xd_ref, w1_ref, b1_ref, w2_ref, b2_ref, o_ref):
    h = jnp.dot(xs_ref[...] * xd_ref[...], w1_ref[...],
                preferred_element_type=jnp.float32) + b1_ref[...]
    h = jnp.maximum(h, 0.0)
    o_ref[...] = jnp.dot(h, w2_ref[...],
                         preferred_element_type=jnp.float32) + b2_ref[...]


def _head_mlp(xs, xd, w1, b1, w2p, b2p):
    return pl.pallas_call(
        _head_mlp_body,
        out_shape=jax.ShapeDtypeStruct((_G, _H), jnp.float32),
    )(xs, xd, w1, b1, w2p, b2p)


def kernel(z, edge_index, batch, z_emb_table, Wl, bl, Wr,
           lin1_W, lin1_b, lin2_W, lin2_b):
    z = z.astype(jnp.int32)
    src = edge_index[0].astype(jnp.int32)
    dst = edge_index[1].astype(jnp.int32)
    batch = batch.astype(jnp.int32)

    npad = _EP - _E
    src_p = jnp.concatenate([src, jnp.zeros((npad,), jnp.int32)])
    src_p = src_p.reshape(_NW, _CPW, _CH)
    dst_p = jnp.concatenate([dst, jnp.full((npad,), _N, jnp.int32)])
    dst_p = dst_p.reshape(_NW, _CPW, _CH)
    z_p = jnp.concatenate([z, jnp.zeros((_NZP - _N,), jnp.int32)])
    batch_p = jnp.concatenate([batch, jnp.full((_NBP - _N,), _G, jnp.int32)])
    batch_p = batch_p.reshape(_NS, _BCW, _CH)

    zeros_h = jnp.zeros((_NACC, _H), jnp.float32)
    ones_h = jnp.ones((_CH, _H), jnp.float32)

    x0 = _zemb_gather(z_emb_table, z_p)
    y, r = _mm2(x0[:_N], Wl[0], Wr[0], bl[0].reshape(1, _H))

    dg, hist = _deg_hist(dst_p, batch_p, zeros_h, ones_h)
    cis, cid = _center_idx(hist)

    p = _segsum(y, src_p, dst_p, zeros_h)
    y, r = _combine_mm(p[0], p[1], dg[0], dg[1], r,
                       Wl[1], Wr[1], bl[1].reshape(1, _H))

    p = _segsum(y, src_p, dst_p, zeros_h)
    y, r = _combine_mm(p[0], p[1], dg[0], dg[1], r,
                       Wl[2], Wr[2], bl[2].reshape(1, _H))

    p = _segsum(y, src_p, dst_p, zeros_h)
    x3 = _final_combine(p[0], p[1], dg[0], dg[1], r)

    xs, xd = _center_gather(x3, cis[:, 0], cid[:, 0])
    w2p = jnp.pad(lin2_W, ((0, 0), (0, _H - 1)))
    b2p = jnp.pad(lin2_b, (0, _H - 1)).reshape(1, _H)
    outp = _head_mlp(xs, xd, lin1_W, lin1_b.reshape(1, _H), w2p, b2p)
    return outp[:, :1]
